# baseline (speedup 1.0000x reference)
import functools

import jax
import jax.numpy as jnp
from jax import lax
from jax.experimental import pallas as pl
from jax.experimental.pallas import tpu as pltpu
from jax.experimental.pallas import tpu_sc as plsc

N_NODES = 10000
N_EDGES = 320000
D = 128
N_LAYERS = 5

NC = 2
NS = 16
NW = NC * NS
EPW = N_EDGES // NW
CHUNK = 40
NCHUNK = EPW // CHUNK
NPT = 624
NREM = N_NODES - NPT * NS
LANES = 16


def _sc_body(x_hbm, src_hbm, dst_hbm, e_hbm, z_hbm, out_hbm,
             sidx, didx, xrows, erows, mrows, agg,
             sem_i, sem_d, sem_e, sem_x, sem_s):
    c = lax.axis_index("c")
    s = lax.axis_index("s")

    pltpu.sync_copy(z_hbm, agg.at[pl.ds(s * NPT, NPT)])

    @pl.when(s == 0)
    def _():
        pltpu.sync_copy(z_hbm.at[pl.ds(0, NREM)],
                        agg.at[pl.ds(NPT * NS, NREM)])

    plsc.subcore_barrier()

    ebase = (c * NS + s) * EPW
    dummy_rows = e_hbm.at[pl.ds(0, CHUNK)]
    dummy_idx = src_hbm.at[pl.ds(0, CHUNK)]

    def issue_idx(n, b2, b3):
        base = ebase + n * CHUNK
        pltpu.async_copy(src_hbm.at[pl.ds(base, CHUNK)], sidx.at[b2],
                         sem_i.at[b2])
        pltpu.async_copy(dst_hbm.at[pl.ds(base, CHUNK)], didx.at[b3],
                         sem_d.at[b3])

    def issue_eattr(n, b2):
        base = ebase + n * CHUNK
        pltpu.async_copy(e_hbm.at[pl.ds(base, CHUNK)], erows.at[b2],
                         sem_e.at[b2])

    def issue_gather(b2):
        pltpu.async_copy(x_hbm.at[sidx.at[b2]], xrows.at[b2], sem_x.at[b2])

    issue_idx(0, 0, 0)
    issue_idx(1, 1, 1)
    issue_eattr(0, 0)
    issue_eattr(1, 1)
    pltpu.make_async_copy(dummy_idx, sidx.at[0], sem_i.at[0]).wait()
    issue_gather(0)

    def outer(k, carry):
        for j in range(6):
            b2 = j % 2
            b3 = j % 3
            cur = 6 * k + j

            @pl.when(cur < NCHUNK)
            def _():
                pltpu.make_async_copy(dummy_rows, erows.at[b2],
                                      sem_e.at[b2]).wait()
                pltpu.make_async_copy(dummy_rows, xrows.at[b2],
                                      sem_x.at[b2]).wait()

                @plsc.parallel_loop(0, CHUNK, unroll=2)
                def _(r):
                    for cc in range(D // LANES):
                        sl = pl.ds(cc * LANES, LANES)
                        mrows[b2, r, sl] = jnp.maximum(
                            erows[b2, r, sl] + xrows[b2, r, sl], 0.0)

                @pl.when(cur >= 1)
                def _():
                    pltpu.make_async_copy(dummy_rows, mrows.at[1 - b2],
                                          sem_s.at[1 - b2]).wait()

                @pl.when(cur + 2 < NCHUNK)
                def _():
                    issue_idx(cur + 2, b2, (j + 2) % 3)
                    issue_eattr(cur + 2, b2)

                @pl.when(cur + 1 < NCHUNK)
                def _():
                    pltpu.make_async_copy(dummy_idx, sidx.at[1 - b2],
                                          sem_i.at[1 - b2]).wait()
                    issue_gather(1 - b2)

                pltpu.async_copy(mrows.at[b2], agg.at[didx.at[b3]],
                                 sem_s.at[b2], add=True)
        return carry

    lax.fori_loop(0, (NCHUNK + 5) // 6, outer, 0)
    pltpu.make_async_copy(dummy_rows, mrows.at[(NCHUNK - 1) % 2],
                          sem_s.at[(NCHUNK - 1) % 2]).wait()

    plsc.subcore_barrier()
    pltpu.sync_copy(agg.at[pl.ds(s * NPT, NPT)],
                    out_hbm.at[c, pl.ds(s * NPT, NPT)])

    @pl.when(s == 0)
    def _():
        pltpu.sync_copy(agg.at[pl.ds(NPT * NS, NREM)],
                        out_hbm.at[c, pl.ds(NPT * NS, NREM)])


_sc_agg = functools.partial(
    pl.kernel,
    out_type=jax.ShapeDtypeStruct((NC, N_NODES, D), jnp.float32),
    mesh=plsc.VectorSubcoreMesh(core_axis_name="c", subcore_axis_name="s",
                                num_cores=NC, num_subcores=NS),
    scratch_types=[
        pltpu.VMEM((2, CHUNK), jnp.int32),
        pltpu.VMEM((3, CHUNK), jnp.int32),
        pltpu.VMEM((2, CHUNK, D), jnp.float32),
        pltpu.VMEM((2, CHUNK, D), jnp.float32),
        pltpu.VMEM((2, CHUNK, D), jnp.float32),
        pltpu.VMEM_SHARED((N_NODES, D), jnp.float32),
        pltpu.SemaphoreType.DMA((2,)),
        pltpu.SemaphoreType.DMA((3,)),
        pltpu.SemaphoreType.DMA((2,)),
        pltpu.SemaphoreType.DMA((2,)),
        pltpu.SemaphoreType.DMA((2,)),
    ],
)(_sc_body)


ROWS_BLK = 2000


def _mlp_body(scale_ref, x_ref, parts_ref, w1_ref, b1_ref, w2_ref, b2_ref,
              out_ref):
    h = (scale_ref[0, 0] * x_ref[...]
         + parts_ref[0, :, :] + parts_ref[1, :, :])
    h = lax.dot_general(h, w1_ref[...], (((1,), (1,)), ((), ())),
                        preferred_element_type=jnp.float32)
    h = jnp.maximum(h + b1_ref[...], 0.0)
    h = lax.dot_general(h, w2_ref[...], (((1,), (1,)), ((), ())),
                        preferred_element_type=jnp.float32)
    out_ref[...] = jnp.maximum(h + b2_ref[...], 0.0)


_mlp = pl.pallas_call(
    _mlp_body,
    grid=(N_NODES // ROWS_BLK,),
    in_specs=[
        pl.BlockSpec((1, 1), lambda i: (0, 0), memory_space=pltpu.SMEM),
        pl.BlockSpec((ROWS_BLK, D), lambda i: (i, 0)),
        pl.BlockSpec((NC, ROWS_BLK, D), lambda i: (0, i, 0)),
        pl.BlockSpec((D, D), lambda i: (0, 0)),
        pl.BlockSpec((1, D), lambda i: (0, 0)),
        pl.BlockSpec((D, D), lambda i: (0, 0)),
        pl.BlockSpec((1, D), lambda i: (0, 0)),
    ],
    out_specs=pl.BlockSpec((ROWS_BLK, D), lambda i: (i, 0)),
    out_shape=jax.ShapeDtypeStruct((N_NODES, D), jnp.float32),
)


def kernel(x, edge_index, edge_attr, W1, b1, W2, b2, eps):
    srci = edge_index[0].astype(jnp.int32)
    dsti = edge_index[1].astype(jnp.int32)
    z = jnp.zeros((NPT, D), jnp.float32)
    for l in range(N_LAYERS):
        parts = _sc_agg(x, srci, dsti, edge_attr, z)
        scale = (1.0 + eps[l]).reshape(1, 1)
        x = _mlp(scale, x, parts, W1[l], b1[l].reshape(1, D),
                 W2[l], b2[l].reshape(1, D))
    return x


# speedup vs baseline: 1.3214x; 1.3214x over previous
import functools

import jax
import jax.numpy as jnp
from jax import lax
from jax.experimental import pallas as pl
from jax.experimental.pallas import tpu as pltpu
from jax.experimental.pallas import tpu_sc as plsc

N_NODES = 10000
N_EDGES = 320000
D = 128
N_LAYERS = 5

NC = 2
NS = 16
NW = NC * NS
EPW = N_EDGES // NW
CHUNK = 40
NCHUNK = EPW // CHUNK
NPT = 624
NREM = N_NODES - NPT * NS
LANES = 16


def _sc_body(x_hbm, src_hbm, dst_hbm, e_hbm, z_hbm, out_hbm,
             sidx, didx, xrows, erows, mrows, agg,
             sem_i, sem_d, sem_e, sem_x, sem_s):
    c = lax.axis_index("c")
    s = lax.axis_index("s")

    pltpu.sync_copy(z_hbm, agg.at[pl.ds(s * NPT, NPT)])

    @pl.when(s == 0)
    def _():
        pltpu.sync_copy(z_hbm.at[pl.ds(0, NREM)],
                        agg.at[pl.ds(NPT * NS, NREM)])

    plsc.subcore_barrier()

    ebase = (c * NS + s) * EPW
    dummy_rows = e_hbm.at[pl.ds(0, CHUNK)]
    dummy_idx = src_hbm.at[pl.ds(0, CHUNK)]

    def issue_idx(n, b2, b4):
        base = ebase + n * CHUNK
        pltpu.async_copy(src_hbm.at[pl.ds(base, CHUNK)], sidx.at[b2],
                         sem_i.at[b2])
        pltpu.async_copy(dst_hbm.at[pl.ds(base, CHUNK)], didx.at[b4],
                         sem_d.at[b4])

    def issue_eattr(n, e3):
        base = ebase + n * CHUNK
        pltpu.async_copy(e_hbm.at[pl.ds(base, CHUNK)], erows.at[e3],
                         sem_e.at[e3])

    def issue_gather(b2):
        pltpu.async_copy(x_hbm.at[sidx.at[b2]], xrows.at[b2], sem_x.at[b2])

    issue_idx(0, 0, 0)
    issue_idx(1, 1, 1)
    issue_eattr(0, 0)
    issue_eattr(1, 1)
    pltpu.make_async_copy(dummy_idx, sidx.at[0], sem_i.at[0]).wait()
    issue_gather(0)

    def outer(k, carry):
        for j in range(12):
            b2 = j % 2
            e3 = j % 3
            b4 = j % 4
            cur = 12 * k + j

            @pl.when(cur < NCHUNK)
            def _():
                pltpu.make_async_copy(dummy_rows, erows.at[e3],
                                      sem_e.at[e3]).wait()
                pltpu.make_async_copy(dummy_rows, xrows.at[b2],
                                      sem_x.at[b2]).wait()

                @pl.when(cur >= 2)
                def _():
                    pltpu.make_async_copy(dummy_rows, mrows.at[b2],
                                          sem_s.at[b2]).wait()

                @pl.when(cur + 2 < NCHUNK)
                def _():
                    issue_idx(cur + 2, b2, (j + 2) % 4)
                    issue_eattr(cur + 2, (j + 2) % 3)

                @pl.when(cur + 1 < NCHUNK)
                def _():
                    pltpu.make_async_copy(dummy_idx, sidx.at[1 - b2],
                                          sem_i.at[1 - b2]).wait()
                    issue_gather(1 - b2)

                @plsc.parallel_loop(0, CHUNK, unroll=2)
                def _(r):
                    for cc in range(D // LANES):
                        sl = pl.ds(cc * LANES, LANES)
                        mrows[b2, r, sl] = jnp.maximum(
                            erows[e3, r, sl] + xrows[b2, r, sl], 0.0)

                pltpu.async_copy(mrows.at[b2], agg.at[didx.at[b4]],
                                 sem_s.at[b2], add=True)
        return carry

    lax.fori_loop(0, (NCHUNK + 11) // 12, outer, 0)
    pltpu.make_async_copy(dummy_rows, mrows.at[(NCHUNK - 1) % 2],
                          sem_s.at[(NCHUNK - 1) % 2]).wait()
    pltpu.make_async_copy(dummy_rows, mrows.at[(NCHUNK - 2) % 2],
                          sem_s.at[(NCHUNK - 2) % 2]).wait()

    plsc.subcore_barrier()
    pltpu.sync_copy(agg.at[pl.ds(s * NPT, NPT)],
                    out_hbm.at[c, pl.ds(s * NPT, NPT)])

    @pl.when(s == 0)
    def _():
        pltpu.sync_copy(agg.at[pl.ds(NPT * NS, NREM)],
                        out_hbm.at[c, pl.ds(NPT * NS, NREM)])


_sc_agg = functools.partial(
    pl.kernel,
    out_type=jax.ShapeDtypeStruct((NC, N_NODES, D), jnp.float32),
    mesh=plsc.VectorSubcoreMesh(core_axis_name="c", subcore_axis_name="s",
                                num_cores=NC, num_subcores=NS),
    scratch_types=[
        pltpu.VMEM((2, CHUNK), jnp.int32),
        pltpu.VMEM((4, CHUNK), jnp.int32),
        pltpu.VMEM((2, CHUNK, D), jnp.float32),
        pltpu.VMEM((3, CHUNK, D), jnp.float32),
        pltpu.VMEM((2, CHUNK, D), jnp.float32),
        pltpu.VMEM_SHARED((N_NODES, D), jnp.float32),
        pltpu.SemaphoreType.DMA((2,)),
        pltpu.SemaphoreType.DMA((4,)),
        pltpu.SemaphoreType.DMA((3,)),
        pltpu.SemaphoreType.DMA((2,)),
        pltpu.SemaphoreType.DMA((2,)),
    ],
)(_sc_body)


ROWS_BLK = 2000


def _mlp_body(scale_ref, x_ref, parts_ref, w1_ref, b1_ref, w2_ref, b2_ref,
              out_ref):
    h = (scale_ref[0, 0] * x_ref[...]
         + parts_ref[0, :, :] + parts_ref[1, :, :])
    h = lax.dot_general(h, w1_ref[...], (((1,), (1,)), ((), ())),
                        preferred_element_type=jnp.float32)
    h = jnp.maximum(h + b1_ref[...], 0.0)
    h = lax.dot_general(h, w2_ref[...], (((1,), (1,)), ((), ())),
                        preferred_element_type=jnp.float32)
    out_ref[...] = jnp.maximum(h + b2_ref[...], 0.0)


_mlp = pl.pallas_call(
    _mlp_body,
    grid=(N_NODES // ROWS_BLK,),
    in_specs=[
        pl.BlockSpec((1, 1), lambda i: (0, 0), memory_space=pltpu.SMEM),
        pl.BlockSpec((ROWS_BLK, D), lambda i: (i, 0)),
        pl.BlockSpec((NC, ROWS_BLK, D), lambda i: (0, i, 0)),
        pl.BlockSpec((D, D), lambda i: (0, 0)),
        pl.BlockSpec((1, D), lambda i: (0, 0)),
        pl.BlockSpec((D, D), lambda i: (0, 0)),
        pl.BlockSpec((1, D), lambda i: (0, 0)),
    ],
    out_specs=pl.BlockSpec((ROWS_BLK, D), lambda i: (i, 0)),
    out_shape=jax.ShapeDtypeStruct((N_NODES, D), jnp.float32),
)


def kernel(x, edge_index, edge_attr, W1, b1, W2, b2, eps):
    srci = edge_index[0].astype(jnp.int32)
    dsti = edge_index[1].astype(jnp.int32)
    z = jnp.zeros((NPT, D), jnp.float32)
    for l in range(N_LAYERS):
        parts = _sc_agg(x, srci, dsti, edge_attr, z)
        scale = (1.0 + eps[l]).reshape(1, 1)
        x = _mlp(scale, x, parts, W1[l], b1[l].reshape(1, D),
                 W2[l], b2[l].reshape(1, D))
    return x


# speedup vs baseline: 1.3246x; 1.0024x over previous
import functools

import jax
import jax.numpy as jnp
from jax import lax
from jax.experimental import pallas as pl
from jax.experimental.pallas import tpu as pltpu
from jax.experimental.pallas import tpu_sc as plsc

N_NODES = 10000
N_EDGES = 320000
D = 128
N_LAYERS = 5

NC = 2
NS = 16
NW = NC * NS
EPW = N_EDGES // NW
CHUNK = 40
NCHUNK = EPW // CHUNK
NPT = 624
NREM = N_NODES - NPT * NS
LANES = 16


def _sc_body(x_hbm, src_hbm, dst_hbm, e_hbm, z_hbm, out_hbm,
             sidx, didx, xrows, erows, mrows, agg,
             sem_i, sem_d, sem_e, sem_x, sem_s):
    c = lax.axis_index("c")
    s = lax.axis_index("s")

    pltpu.sync_copy(z_hbm, agg.at[pl.ds(s * NPT, NPT)])

    @pl.when(s == 0)
    def _():
        pltpu.sync_copy(z_hbm.at[pl.ds(0, NREM)],
                        agg.at[pl.ds(NPT * NS, NREM)])

    plsc.subcore_barrier()

    ebase = (c * NS + s) * EPW
    dummy_rows = e_hbm.at[pl.ds(0, CHUNK)]
    dummy_idx = src_hbm.at[pl.ds(0, CHUNK)]

    def issue_idx(n, b2, b4):
        base = ebase + n * CHUNK
        pltpu.async_copy(src_hbm.at[pl.ds(base, CHUNK)], sidx.at[b2],
                         sem_i.at[b2])
        pltpu.async_copy(dst_hbm.at[pl.ds(base, CHUNK)], didx.at[b4],
                         sem_d.at[b4])

    def issue_eattr(n, e3):
        base = ebase + n * CHUNK
        pltpu.async_copy(e_hbm.at[pl.ds(base, CHUNK)], erows.at[e3],
                         sem_e.at[e3])

    def issue_gather(b2):
        pltpu.async_copy(x_hbm.at[sidx.at[b2]], xrows.at[b2], sem_x.at[b2])

    issue_idx(0, 0, 0)
    issue_idx(1, 1, 1)
    issue_eattr(0, 0)
    issue_eattr(1, 1)
    pltpu.make_async_copy(dummy_idx, sidx.at[0], sem_i.at[0]).wait()
    issue_gather(0)

    def outer(k, carry):
        for j in range(12):
            b2 = j % 2
            e3 = j % 3
            b4 = j % 4
            cur = 12 * k + j

            @pl.when(cur < NCHUNK)
            def _():
                pltpu.make_async_copy(dummy_rows, erows.at[e3],
                                      sem_e.at[e3]).wait()
                pltpu.make_async_copy(dummy_rows, xrows.at[b2],
                                      sem_x.at[b2]).wait()

                @pl.when(cur >= 2)
                def _():
                    pltpu.make_async_copy(dummy_rows, mrows.at[b2],
                                          sem_s.at[b2]).wait()

                @pl.when(cur + 2 < NCHUNK)
                def _():
                    issue_idx(cur + 2, b2, (j + 2) % 4)
                    issue_eattr(cur + 2, (j + 2) % 3)

                @pl.when(cur + 1 < NCHUNK)
                def _():
                    pltpu.make_async_copy(dummy_idx, sidx.at[1 - b2],
                                          sem_i.at[1 - b2]).wait()
                    issue_gather(1 - b2)

                @plsc.parallel_loop(0, CHUNK, unroll=2)
                def _(r):
                    for cc in range(D // LANES):
                        sl = pl.ds(cc * LANES, LANES)
                        mrows[b2, r, sl] = jnp.maximum(
                            erows[e3, r, sl] + xrows[b2, r, sl], 0.0)

                pltpu.make_async_copy(dummy_idx, didx.at[b4],
                                      sem_d.at[b4]).wait()
                pltpu.async_copy(mrows.at[b2], agg.at[didx.at[b4]],
                                 sem_s.at[b2], add=True)
        return carry

    lax.fori_loop(0, (NCHUNK + 11) // 12, outer, 0)
    pltpu.make_async_copy(dummy_rows, mrows.at[(NCHUNK - 1) % 2],
                          sem_s.at[(NCHUNK - 1) % 2]).wait()
    pltpu.make_async_copy(dummy_rows, mrows.at[(NCHUNK - 2) % 2],
                          sem_s.at[(NCHUNK - 2) % 2]).wait()

    plsc.subcore_barrier()
    pltpu.sync_copy(agg.at[pl.ds(s * NPT, NPT)],
                    out_hbm.at[c, pl.ds(s * NPT, NPT)])

    @pl.when(s == 0)
    def _():
        pltpu.sync_copy(agg.at[pl.ds(NPT * NS, NREM)],
                        out_hbm.at[c, pl.ds(NPT * NS, NREM)])


_sc_agg = functools.partial(
    pl.kernel,
    out_type=jax.ShapeDtypeStruct((NC, N_NODES, D), jnp.float32),
    mesh=plsc.VectorSubcoreMesh(core_axis_name="c", subcore_axis_name="s",
                                num_cores=NC, num_subcores=NS),
    scratch_types=[
        pltpu.VMEM((2, CHUNK), jnp.int32),
        pltpu.VMEM((4, CHUNK), jnp.int32),
        pltpu.VMEM((2, CHUNK, D), jnp.float32),
        pltpu.VMEM((3, CHUNK, D), jnp.float32),
        pltpu.VMEM((2, CHUNK, D), jnp.float32),
        pltpu.VMEM_SHARED((N_NODES, D), jnp.float32),
        pltpu.SemaphoreType.DMA((2,)),
        pltpu.SemaphoreType.DMA((4,)),
        pltpu.SemaphoreType.DMA((3,)),
        pltpu.SemaphoreType.DMA((2,)),
        pltpu.SemaphoreType.DMA((2,)),
    ],
)(_sc_body)


ROWS_BLK = 2000


def _mlp_body(scale_ref, x_ref, parts_ref, w1_ref, b1_ref, w2_ref, b2_ref,
              out_ref):
    h = (scale_ref[0, 0] * x_ref[...]
         + parts_ref[0, :, :] + parts_ref[1, :, :])
    h = lax.dot_general(h, w1_ref[...], (((1,), (1,)), ((), ())),
                        preferred_element_type=jnp.float32)
    h = jnp.maximum(h + b1_ref[...], 0.0)
    h = lax.dot_general(h, w2_ref[...], (((1,), (1,)), ((), ())),
                        preferred_element_type=jnp.float32)
    out_ref[...] = jnp.maximum(h + b2_ref[...], 0.0)


_mlp = pl.pallas_call(
    _mlp_body,
    grid=(N_NODES // ROWS_BLK,),
    in_specs=[
        pl.BlockSpec((1, 1), lambda i: (0, 0), memory_space=pltpu.SMEM),
        pl.BlockSpec((ROWS_BLK, D), lambda i: (i, 0)),
        pl.BlockSpec((NC, ROWS_BLK, D), lambda i: (0, i, 0)),
        pl.BlockSpec((D, D), lambda i: (0, 0)),
        pl.BlockSpec((1, D), lambda i: (0, 0)),
        pl.BlockSpec((D, D), lambda i: (0, 0)),
        pl.BlockSpec((1, D), lambda i: (0, 0)),
    ],
    out_specs=pl.BlockSpec((ROWS_BLK, D), lambda i: (i, 0)),
    out_shape=jax.ShapeDtypeStruct((N_NODES, D), jnp.float32),
)


def kernel(x, edge_index, edge_attr, W1, b1, W2, b2, eps):
    srci = edge_index[0].astype(jnp.int32)
    dsti = edge_index[1].astype(jnp.int32)
    z = jnp.zeros((NPT, D), jnp.float32)
    for l in range(N_LAYERS):
        parts = _sc_agg(x, srci, dsti, edge_attr, z)
        scale = (1.0 + eps[l]).reshape(1, 1)
        x = _mlp(scale, x, parts, W1[l], b1[l].reshape(1, D),
                 W2[l], b2[l].reshape(1, D))
    return x


# speedup vs baseline: 1.7664x; 1.3335x over previous
import functools

import jax
import jax.numpy as jnp
from jax import lax
from jax.experimental import pallas as pl
from jax.experimental.pallas import tpu as pltpu
from jax.experimental.pallas import tpu_sc as plsc

N_NODES = 10000
N_EDGES = 320000
D = 128
N_LAYERS = 5

NC = 2
NS = 16
NW = NC * NS
EPW = N_EDGES // NW
CHUNK = 64
NCHUNK = EPW // CHUNK
TAIL = EPW - NCHUNK * CHUNK
NPT = 624
NREM = N_NODES - NPT * NS
LANES = 16


def _sc_body(x_hbm, src_hbm, dst_hbm, e_hbm, z_hbm, out_hbm,
             sidx, didx, tidx, xrows, erows, mrows, agg,
             sem_i, sem_d, sem_e, sem_x, sem_s):
    c = lax.axis_index("c")
    s = lax.axis_index("s")

    pltpu.sync_copy(z_hbm, agg.at[pl.ds(s * NPT, NPT)])

    @pl.when(s == 0)
    def _():
        pltpu.sync_copy(z_hbm.at[pl.ds(0, NREM)],
                        agg.at[pl.ds(NPT * NS, NREM)])

    plsc.subcore_barrier()

    ebase = (c * NS + s) * EPW
    dummy_rows = e_hbm.at[pl.ds(0, CHUNK)]
    dummy_idx = src_hbm.at[pl.ds(0, CHUNK)]

    def issue_idx(n, b2, b4):
        base = ebase + n * CHUNK
        pltpu.async_copy(src_hbm.at[pl.ds(base, CHUNK)], sidx.at[b2],
                         sem_i.at[b2])
        pltpu.async_copy(dst_hbm.at[pl.ds(base, CHUNK)], didx.at[b4],
                         sem_d.at[b4])

    def issue_eattr(n, b2):
        base = ebase + n * CHUNK
        pltpu.async_copy(e_hbm.at[pl.ds(base, CHUNK)], erows.at[b2],
                         sem_e.at[b2])

    def issue_gather(b2):
        pltpu.async_copy(x_hbm.at[sidx.at[b2]], xrows.at[b2], sem_x.at[b2])

    issue_idx(0, 0, 0)
    issue_idx(1, 1, 1)
    issue_eattr(0, 0)
    issue_eattr(1, 1)
    pltpu.make_async_copy(dummy_idx, sidx.at[0], sem_i.at[0]).wait()
    issue_gather(0)

    def outer(k, carry):
        for j in range(4):
            b2 = j % 2
            b4 = j % 4
            cur = 4 * k + j

            pltpu.make_async_copy(dummy_rows, erows.at[b2],
                                  sem_e.at[b2]).wait()
            pltpu.make_async_copy(dummy_rows, xrows.at[b2],
                                  sem_x.at[b2]).wait()

            @pl.when(cur >= 2)
            def _():
                pltpu.make_async_copy(dummy_rows, mrows.at[b2],
                                      sem_s.at[b2]).wait()

            @pl.when(cur + 2 < NCHUNK)
            def _():
                issue_idx(cur + 2, b2, (j + 2) % 4)

            @pl.when(cur + 1 < NCHUNK)
            def _():
                pltpu.make_async_copy(dummy_idx, sidx.at[1 - b2],
                                      sem_i.at[1 - b2]).wait()
                issue_gather(1 - b2)

            @plsc.parallel_loop(0, CHUNK, unroll=2)
            def _(r):
                for cc in range(D // LANES):
                    sl = pl.ds(cc * LANES, LANES)
                    mrows[b2, r, sl] = jnp.maximum(
                        erows[b2, r, sl] + xrows[b2, r, sl], 0.0)

            @pl.when(cur + 2 < NCHUNK)
            def _():
                issue_eattr(cur + 2, b2)

            pltpu.make_async_copy(dummy_idx, didx.at[b4],
                                  sem_d.at[b4]).wait()
            pltpu.async_copy(mrows.at[b2], agg.at[didx.at[b4]],
                             sem_s.at[b2], add=True)
        return carry

    lax.fori_loop(0, NCHUNK // 4, outer, 0)
    pltpu.make_async_copy(dummy_rows, mrows.at[(NCHUNK - 1) % 2],
                          sem_s.at[(NCHUNK - 1) % 2]).wait()
    pltpu.make_async_copy(dummy_rows, mrows.at[(NCHUNK - 2) % 2],
                          sem_s.at[(NCHUNK - 2) % 2]).wait()

    tb = ebase + NCHUNK * CHUNK
    pltpu.sync_copy(src_hbm.at[pl.ds(tb, TAIL)], tidx.at[0])
    pltpu.sync_copy(dst_hbm.at[pl.ds(tb, TAIL)], tidx.at[1])
    pltpu.sync_copy(e_hbm.at[pl.ds(tb, TAIL)], erows.at[0, pl.ds(0, TAIL)])
    pltpu.async_copy(x_hbm.at[tidx.at[0]], xrows.at[0, pl.ds(0, TAIL)],
                     sem_x.at[0]).wait()

    @plsc.parallel_loop(0, TAIL, unroll=2)
    def _(r):
        for cc in range(D // LANES):
            sl = pl.ds(cc * LANES, LANES)
            mrows[0, r, sl] = jnp.maximum(
                erows[0, r, sl] + xrows[0, r, sl], 0.0)

    pltpu.sync_copy(mrows.at[0, pl.ds(0, TAIL)], agg.at[tidx.at[1]],
                    add=True)

    plsc.subcore_barrier()
    pltpu.sync_copy(agg.at[pl.ds(s * NPT, NPT)],
                    out_hbm.at[c, pl.ds(s * NPT, NPT)])

    @pl.when(s == 0)
    def _():
        pltpu.sync_copy(agg.at[pl.ds(NPT * NS, NREM)],
                        out_hbm.at[c, pl.ds(NPT * NS, NREM)])


_sc_agg = functools.partial(
    pl.kernel,
    out_type=jax.ShapeDtypeStruct((NC, N_NODES, D), jnp.float32),
    mesh=plsc.VectorSubcoreMesh(core_axis_name="c", subcore_axis_name="s",
                                num_cores=NC, num_subcores=NS),
    scratch_types=[
        pltpu.VMEM((2, CHUNK), jnp.int32),
        pltpu.VMEM((4, CHUNK), jnp.int32),
        pltpu.VMEM((2, TAIL), jnp.int32),
        pltpu.VMEM((2, CHUNK, D), jnp.float32),
        pltpu.VMEM((2, CHUNK, D), jnp.float32),
        pltpu.VMEM((2, CHUNK, D), jnp.float32),
        pltpu.VMEM_SHARED((N_NODES, D), jnp.float32),
        pltpu.SemaphoreType.DMA((2,)),
        pltpu.SemaphoreType.DMA((4,)),
        pltpu.SemaphoreType.DMA((2,)),
        pltpu.SemaphoreType.DMA((2,)),
        pltpu.SemaphoreType.DMA((2,)),
    ],
)(_sc_body)


ROWS_BLK = 2000


def _mlp_body(scale_ref, x_ref, parts_ref, w1_ref, b1_ref, w2_ref, b2_ref,
              out_ref):
    h = (scale_ref[0, 0] * x_ref[...]
         + parts_ref[0, :, :] + parts_ref[1, :, :])
    h = lax.dot_general(h, w1_ref[...], (((1,), (1,)), ((), ())),
                        preferred_element_type=jnp.float32)
    h = jnp.maximum(h + b1_ref[...], 0.0)
    h = lax.dot_general(h, w2_ref[...], (((1,), (1,)), ((), ())),
                        preferred_element_type=jnp.float32)
    out_ref[...] = jnp.maximum(h + b2_ref[...], 0.0)


_mlp = pl.pallas_call(
    _mlp_body,
    grid=(N_NODES // ROWS_BLK,),
    in_specs=[
        pl.BlockSpec((1, 1), lambda i: (0, 0), memory_space=pltpu.SMEM),
        pl.BlockSpec((ROWS_BLK, D), lambda i: (i, 0)),
        pl.BlockSpec((NC, ROWS_BLK, D), lambda i: (0, i, 0)),
        pl.BlockSpec((D, D), lambda i: (0, 0)),
        pl.BlockSpec((1, D), lambda i: (0, 0)),
        pl.BlockSpec((D, D), lambda i: (0, 0)),
        pl.BlockSpec((1, D), lambda i: (0, 0)),
    ],
    out_specs=pl.BlockSpec((ROWS_BLK, D), lambda i: (i, 0)),
    out_shape=jax.ShapeDtypeStruct((N_NODES, D), jnp.float32),
)


def kernel(x, edge_index, edge_attr, W1, b1, W2, b2, eps):
    srci = edge_index[0].astype(jnp.int32)
    dsti = edge_index[1].astype(jnp.int32)
    z = jnp.zeros((NPT, D), jnp.float32)
    for l in range(N_LAYERS):
        parts = _sc_agg(x, srci, dsti, edge_attr, z)
        scale = (1.0 + eps[l]).reshape(1, 1)
        x = _mlp(scale, x, parts, W1[l], b1[l].reshape(1, D),
                 W2[l], b2[l].reshape(1, D))
    return x
